# baseline (speedup 1.0000x reference)
import functools

import jax
import jax.numpy as jnp
from jax.experimental import pallas as pl
from jax.experimental.pallas import tpu as pltpu
from jax.experimental.pallas import tpu_sc as plsc

_BLK = 256
_GW = 16
_NBUF = 3


def _routing(idx, n, num_experts, blk, m):
    e_range = jnp.arange(num_experts, dtype=jnp.int32)
    onehot = (idx[:, None] == e_range[None, :]).astype(jnp.int32)
    csum = jnp.cumsum(onehot, axis=0)
    counts = csum[-1]
    rank = jnp.take_along_axis(csum, idx[:, None], axis=1)[:, 0] - 1
    padded = ((counts + blk - 1) // blk) * blk
    ends = jnp.cumsum(padded)
    starts = ends - padded
    dst = starts[idx] + rank
    blk_base = jnp.arange(m // blk, dtype=jnp.int32) * blk
    block_expert = jnp.sum(
        (blk_base[:, None] >= ends[None, :]).astype(jnp.int32), axis=1)
    block_expert = jnp.minimum(block_expert, num_experts - 1)
    return dst, block_expert


def _sc_row_gather(table, idx):
    q = idx.shape[0]
    d = table.shape[1]
    mesh = plsc.VectorSubcoreMesh(core_axis_name="core",
                                  subcore_axis_name="subcore")
    num_workers = mesh.num_cores * mesh.num_subcores
    rows_per_w = q // num_workers
    nch = rows_per_w // _GW

    @functools.partial(
        pl.kernel,
        out_type=jax.ShapeDtypeStruct((q, d), table.dtype),
        mesh=mesh,
        scratch_types=[
            pltpu.VMEM((rows_per_w,), jnp.int32),
            pltpu.VMEM((_NBUF, _GW, d), table.dtype),
            pltpu.SemaphoreType.DMA((_NBUF,)),
            pltpu.SemaphoreType.DMA((_NBUF,)),
        ],
    )
    def gather_kernel(x_hbm, i_hbm, o_hbm, idx_v, bufs, isems, osems):
        wid = (jax.lax.axis_index("subcore") * mesh.num_cores
               + jax.lax.axis_index("core"))
        base = wid * rows_per_w
        pltpu.sync_copy(i_hbm.at[pl.ds(base, rows_per_w)], idx_v)

        def read(c):
            return pltpu.async_copy(
                x_hbm.at[idx_v.at[pl.ds(c * _GW, _GW)]],
                bufs.at[c % _NBUF], isems.at[c % _NBUF])

        def write(c):
            return pltpu.async_copy(
                bufs.at[c % _NBUF], o_hbm.at[pl.ds(base + c * _GW, _GW)],
                osems.at[c % _NBUF])

        reads, writes = {}, {}
        reads[0] = read(0)
        for c in range(nch):
            reads[c].wait()
            if c + 1 < nch:
                if c + 1 >= _NBUF:
                    writes[c + 1 - _NBUF].wait()
                reads[c + 1] = read(c + 1)
            writes[c] = write(c)
        for c in range(max(0, nch - _NBUF + 1), nch):
            writes[c].wait()

    return gather_kernel(table, idx)


def _sc_row_scatter(rows, idx, q, row_offset=0):
    d = rows.shape[1]
    nsc = idx.shape[0]
    mesh = plsc.VectorSubcoreMesh(core_axis_name="core",
                                  subcore_axis_name="subcore")
    num_workers = mesh.num_cores * mesh.num_subcores
    rows_per_w = nsc // num_workers
    nch = rows_per_w // _GW
    idx3d = idx.reshape((num_workers, nch, _GW))

    @functools.partial(
        pl.kernel,
        out_type=jax.ShapeDtypeStruct((q, d), rows.dtype),
        mesh=mesh,
        scratch_types=[
            pltpu.VMEM((nch, _GW), jnp.int32),
            pltpu.VMEM((_NBUF, _GW, d), rows.dtype),
            pltpu.SemaphoreType.DMA((_NBUF,)),
            pltpu.SemaphoreType.DMA((_NBUF,)),
        ],
    )
    def scatter_kernel(x_hbm, i_hbm, o_hbm, idx_v, bufs, isems, osems):
        wid = (jax.lax.axis_index("subcore") * mesh.num_cores
               + jax.lax.axis_index("core"))
        base = row_offset + wid * rows_per_w
        pltpu.sync_copy(i_hbm.at[wid], idx_v)

        def read(c):
            return pltpu.async_copy(
                x_hbm.at[pl.ds(base + c * _GW, _GW)],
                bufs.at[c % _NBUF], isems.at[c % _NBUF])

        def write(c):
            return pltpu.async_copy(
                bufs.at[c % _NBUF], o_hbm.at[idx_v.at[c]],
                osems.at[c % _NBUF])

        reads, writes = {}, {}
        reads[0] = read(0)
        for c in range(nch):
            reads[c].wait()
            if c + 1 < nch:
                if c + 1 >= _NBUF:
                    writes[c + 1 - _NBUF].wait()
                reads[c + 1] = read(c + 1)
            writes[c] = write(c)
        for c in range(max(0, nch - _NBUF + 1), nch):
            writes[c].wait()

    return scatter_kernel(rows, idx3d)


def _mm_body(meta_ref, xs_ref, w_ref, b_ref, o_ref, wbf_ref):
    i = pl.program_id(0)

    @pl.when(meta_ref[i, 1] == 1)
    def _():
        wbf_ref[meta_ref[i, 2]] = w_ref[0].astype(jnp.bfloat16)

    @pl.when(i > 0)
    def _():
        acc = jnp.dot(xs_ref[...].astype(jnp.bfloat16),
                      wbf_ref[meta_ref[jnp.maximum(i - 1, 0), 2]],
                      preferred_element_type=jnp.float32)
        o_ref[...] = jnp.maximum(acc + b_ref[0], 0.0)


def _expert_matmul(xs, w, b, block_expert, m, d):
    num_blocks = m // _BLK
    be_look = jnp.concatenate([block_expert, block_expert[-1:]])
    cast_flag = jnp.concatenate(
        [jnp.ones((1,), jnp.int32),
         (be_look[1:] != be_look[:-1]).astype(jnp.int32)])
    cast_buf = (jnp.cumsum(cast_flag) - 1) % 2
    meta = jnp.stack([be_look, cast_flag, cast_buf], axis=1)

    grid_spec = pltpu.PrefetchScalarGridSpec(
        num_scalar_prefetch=1,
        grid=(num_blocks + 1,),
        in_specs=[
            pl.BlockSpec(
                (_BLK, d),
                lambda i, mref: (jnp.maximum(i - 1, 0), 0)),
            pl.BlockSpec(
                (1, d, d),
                lambda i, mref: (mref[i, 0], 0, 0)),
            pl.BlockSpec(
                (1, 1, d),
                lambda i, mref: (mref[jnp.maximum(i - 1, 0), 0], 0, 0)),
        ],
        out_specs=pl.BlockSpec(
            (_BLK, d), lambda i, mref: (jnp.maximum(i - 1, 0), 0)),
        scratch_shapes=[pltpu.VMEM((2, d, d), jnp.bfloat16)],
    )
    return pl.pallas_call(
        _mm_body,
        grid_spec=grid_spec,
        out_shape=jax.ShapeDtypeStruct((m, d), jnp.float32),
        compiler_params=pltpu.CompilerParams(
            dimension_semantics=("arbitrary",)),
    )(meta, xs, w, b.reshape(b.shape[0], 1, d))


def kernel(x, groups, W, b):
    n, d = x.shape
    num_experts = W.shape[0]
    m = n + num_experts * _BLK

    idx = groups[:, 0].astype(jnp.int32)
    dst, block_expert = _routing(idx, n, num_experts, _BLK, m)

    xs = _sc_row_scatter(x, dst, m)
    ys = _expert_matmul(xs, W, b, block_expert, m, d)
    return _sc_row_gather(ys, dst)
